# baseline (speedup 1.0000x reference)
import functools

import jax
import jax.numpy as jnp
from jax import lax
from jax.experimental import pallas as pl
from jax.experimental.pallas import tpu as pltpu
from jax.experimental.pallas import tpu_sc as plsc

_NC = 2
_NS = 16
_NW = _NC * _NS

_GB = 4


def _body(table_hbm, idx_hbm, out_hbm, idx_v, rows_v, gsem, osem0, osem1):
    seq = idx_hbm.shape[1]
    wid = lax.axis_index("s") * _NC + lax.axis_index("c")
    nrows = idx_hbm.shape[0] // _NW
    b0 = wid * nrows
    ng = nrows // _GB
    osems = (osem0, osem1)
    splits = [(0, 128), (128, seq - 128)] if seq > 128 else [(0, seq)]

    pltpu.sync_copy(idx_hbm.at[pl.ds(b0, _GB)], idx_v.at[0])

    @pl.loop(0, ng, step=2)
    def _pair(p):
        for b in range(2):
            g = p + b
            row0 = b0 + g * _GB
            rows = rows_v.at[b]
            out_slice = out_hbm.at[pl.ds(row0, _GB)]

            @pl.when(g >= 2)
            def _():
                pltpu.make_async_copy(rows, out_slice, osems[b]).wait()

            copies = [
                pltpu.async_copy(
                    table_hbm.at[idx_v.at[b, r, pl.ds(lo, ln)]],
                    rows.at[r, pl.ds(lo, ln)],
                    gsem,
                )
                for r in range(_GB)
                for (lo, ln) in splits
            ]

            @pl.when(g + 1 < ng)
            def _():
                pltpu.sync_copy(
                    idx_hbm.at[pl.ds(row0 + _GB, _GB)], idx_v.at[1 - b]
                )

            for cp in copies:
                cp.wait()

            pltpu.async_copy(rows, out_slice, osems[b])

    for b in range(2):
        pltpu.make_async_copy(
            rows_v.at[b], out_hbm.at[pl.ds(b0, _GB)], osems[b]
        ).wait()


@jax.jit
def _gather(averages, idx2d):
    batch, seq = idx2d.shape
    d = averages.shape[1]
    mesh = plsc.VectorSubcoreMesh(core_axis_name="c", subcore_axis_name="s")
    return pl.kernel(
        _body,
        out_type=jax.ShapeDtypeStruct((batch, seq, d), averages.dtype),
        mesh=mesh,
        scratch_types=[
            pltpu.VMEM((2, _GB, seq), jnp.int32),
            pltpu.VMEM((2, _GB, seq, d), jnp.float32),
            pltpu.SemaphoreType.DMA,
            pltpu.SemaphoreType.DMA,
            pltpu.SemaphoreType.DMA,
        ],
        compiler_params=pltpu.CompilerParams(use_tc_tiling_on_sc=False),
    )(averages, idx2d)


def kernel(indicies, averages):
    return _gather(averages, indicies.astype(jnp.int32))


# speedup vs baseline: 1.6529x; 1.6529x over previous
import functools

import jax
import jax.numpy as jnp
from jax import lax
from jax.experimental import pallas as pl
from jax.experimental.pallas import tpu as pltpu
from jax.experimental.pallas import tpu_sc as plsc

_NC = 2
_NS = 16
_NW = _NC * _NS

_GB = 4


def _body(table_hbm, idx_hbm, out_hbm, idx_v, rows_v, gsem, osem0, osem1):
    seq = idx_hbm.shape[1]
    wid = lax.axis_index("s") * _NC + lax.axis_index("c")
    nrows = idx_hbm.shape[0] // _NW
    b0 = wid * nrows
    ng = nrows // _GB
    osems = (osem0, osem1)
    splits = [(0, 128), (128, seq - 128)] if seq > 128 else [(0, seq)]

    pltpu.sync_copy(idx_hbm.at[pl.ds(b0, _GB)], idx_v.at[0])

    @pl.loop(0, ng, step=2)
    def _pair(p):
        for b in range(2):
            g = p + b
            row0 = b0 + g * _GB
            rows = rows_v.at[b]
            out_slice = out_hbm.at[pl.ds(row0, _GB), :, pl.ds(0, 64)]

            @pl.when(g >= 2)
            def _():
                pltpu.make_async_copy(rows, out_slice, osems[b]).wait()

            copies = [
                pltpu.async_copy(
                    table_hbm.at[idx_v.at[b, r, pl.ds(lo, ln)]],
                    rows.at[r, pl.ds(lo, ln)],
                    gsem,
                )
                for r in range(_GB)
                for (lo, ln) in splits
            ]

            @pl.when(g + 1 < ng)
            def _():
                pltpu.sync_copy(
                    idx_hbm.at[pl.ds(row0 + _GB, _GB)], idx_v.at[1 - b]
                )

            for cp in copies:
                cp.wait()

            pltpu.async_copy(rows, out_slice, osems[b])

    for b in range(2):
        pltpu.make_async_copy(
            rows_v.at[b], out_hbm.at[pl.ds(b0, _GB), :, pl.ds(0, 64)], osems[b]
        ).wait()


@jax.jit
def _gather(averages, idx2d):
    batch, seq = idx2d.shape
    d = averages.shape[1]
    mesh = plsc.VectorSubcoreMesh(core_axis_name="c", subcore_axis_name="s")
    return pl.kernel(
        _body,
        out_type=jax.ShapeDtypeStruct((batch, seq, 128), averages.dtype),
        mesh=mesh,
        scratch_types=[
            pltpu.VMEM((2, _GB, seq), jnp.int32),
            pltpu.VMEM((2, _GB, seq, d), jnp.float32),
            pltpu.SemaphoreType.DMA,
            pltpu.SemaphoreType.DMA,
            pltpu.SemaphoreType.DMA,
        ],
        compiler_params=pltpu.CompilerParams(use_tc_tiling_on_sc=False),
    )(averages, idx2d)


def kernel(indicies, averages):
    return _gather(averages, indicies.astype(jnp.int32))[..., :64]


# speedup vs baseline: 1.6543x; 1.0008x over previous
import functools

import jax
import jax.numpy as jnp
from jax import lax
from jax.experimental import pallas as pl
from jax.experimental.pallas import tpu as pltpu
from jax.experimental.pallas import tpu_sc as plsc

_NC = 2
_NS = 16
_NW = _NC * _NS

_GB = 4


def _body(table_hbm, idx_hbm, out_hbm, idx_v, rows_v, gsem, osem0, osem1):
    seq = idx_hbm.shape[1]
    wid = lax.axis_index("s") * _NC + lax.axis_index("c")
    nrows = idx_hbm.shape[0] // _NW
    b0 = wid * nrows
    ng = nrows // _GB
    osems = (osem0, osem1)
    splits = [(0, 128), (128, seq - 128)] if seq > 128 else [(0, seq)]

    def fire_gathers(b, row0):
        return [
            pltpu.async_copy(
                table_hbm.at[idx_v.at[b, r, pl.ds(lo, ln)]],
                rows_v.at[b, r, pl.ds(lo, ln)],
                gsem,
            )
            for r in range(_GB)
            for (lo, ln) in splits
        ]

    def drain_gathers(b):
        for r in range(_GB):
            for (lo, ln) in splits:
                pltpu.make_async_copy(
                    table_hbm.at[idx_v.at[b, r, pl.ds(lo, ln)]],
                    rows_v.at[b, r, pl.ds(lo, ln)],
                    gsem,
                ).wait()

    def out_slice(row0):
        return out_hbm.at[pl.ds(row0, _GB), :, pl.ds(0, 64)]

    pltpu.sync_copy(idx_hbm.at[pl.ds(b0, _GB)], idx_v.at[0])
    fire_gathers(0, b0)
    pltpu.sync_copy(idx_hbm.at[pl.ds(b0 + _GB, _GB)], idx_v.at[1])

    @pl.loop(0, ng, step=2)
    def _pair(p):
        for b in range(2):
            g = p + b
            row0 = b0 + g * _GB

            @pl.when(g >= 1)
            def _():
                pltpu.make_async_copy(
                    rows_v.at[1 - b], out_slice(row0 - _GB), osems[1 - b]
                ).wait()

            @pl.when(g + 1 < ng)
            def _():
                fire_gathers(1 - b, row0 + _GB)

            drain_gathers(b)
            pltpu.async_copy(rows_v.at[b], out_slice(row0), osems[b])

            @pl.when(g + 2 < ng)
            def _():
                pltpu.sync_copy(
                    idx_hbm.at[pl.ds(row0 + 2 * _GB, _GB)], idx_v.at[b]
                )

    bl = (ng - 1) % 2
    pltpu.make_async_copy(
        rows_v.at[bl], out_hbm.at[pl.ds(b0, _GB), :, pl.ds(0, 64)], osems[bl]
    ).wait()


@jax.jit
def _gather(averages, idx2d):
    batch, seq = idx2d.shape
    d = averages.shape[1]
    mesh = plsc.VectorSubcoreMesh(core_axis_name="c", subcore_axis_name="s")
    return pl.kernel(
        _body,
        out_type=jax.ShapeDtypeStruct((batch, seq, 128), averages.dtype),
        mesh=mesh,
        scratch_types=[
            pltpu.VMEM((2, _GB, seq), jnp.int32),
            pltpu.VMEM((2, _GB, seq, d), jnp.float32),
            pltpu.SemaphoreType.DMA,
            pltpu.SemaphoreType.DMA,
            pltpu.SemaphoreType.DMA,
        ],
        compiler_params=pltpu.CompilerParams(use_tc_tiling_on_sc=False),
    )(averages, idx2d)


def kernel(indicies, averages):
    return _gather(averages, indicies.astype(jnp.int32))[..., :64]
